# baseline (speedup 1.0000x reference)
import functools

import jax
import jax.numpy as jnp
from jax import lax
from jax.experimental import pallas as pl
from jax.experimental.pallas import tpu as pltpu
from jax.experimental.pallas import tpu_sc as plsc

_N = 10000
_E = 320000
_D_IN = 128

_NP = 10240
_ROWS_PER_TILE = _NP // 16
_NW = 32
_CH = 512
_NBUF = 2
_NCHUNK = _NBUF * (-(-_E // (_NW * _CH * _NBUF)))
_EPAD = _NW * _NCHUNK * _CH
_BLK = 1024
_GRID = _NP // _BLK


def _sc_scatter_fn(width, nchunk):
    mesh = plsc.VectorSubcoreMesh(
        core_axis_name="c", subcore_axis_name="s", num_cores=2, num_subcores=16
    )

    @functools.partial(
        pl.kernel,
        out_type=jax.ShapeDtypeStruct((2, _NP, width), jnp.float32),
        mesh=mesh,
        scratch_types=[
            pltpu.VMEM((nchunk, _CH), jnp.int32),
            pltpu.VMEM((nchunk, _CH), jnp.int32),
            pltpu.VMEM((_NBUF, _CH, width), jnp.float32),
            pltpu.VMEM_SHARED((_NP, width), jnp.float32),
        ] + [pltpu.SemaphoreType.DMA] * _NBUF,
        compiler_params=pltpu.CompilerParams(use_tc_tiling_on_sc=False),
    )
    def sc_kernel(g_hbm, src_hbm, dst_hbm, zeros_hbm, out_hbm,
                  src_v, dst_v, rows_v, acc_sh, *sems):
        cid = lax.axis_index("c")
        sid = lax.axis_index("s")
        wid = sid * 2 + cid
        row0 = sid * _ROWS_PER_TILE
        pltpu.sync_copy(zeros_hbm, acc_sh.at[pl.ds(row0, _ROWS_PER_TILE)])
        pltpu.sync_copy(src_hbm.at[wid], src_v)
        pltpu.sync_copy(dst_hbm.at[wid], dst_v)
        plsc.subcore_barrier()

        def body(j, carry):
            pltpu.async_copy(g_hbm.at[src_v.at[j]], rows_v.at[0], sems[0]).wait()
            pltpu.sync_copy(rows_v.at[0], acc_sh.at[dst_v.at[j]], add=True)
            return carry

        lax.fori_loop(0, nchunk, body, 0)
        plsc.subcore_barrier()
        pltpu.sync_copy(acc_sh.at[pl.ds(row0, _ROWS_PER_TILE)],
                        out_hbm.at[cid, pl.ds(row0, _ROWS_PER_TILE)])

    return sc_kernel


def _tc1_body(x_ref, wg_ref, cg_ref, wr_ref, br_ref, g1_ref, xr1_ref):
    xb = x_ref[...]
    g1_ref[...] = jnp.dot(xb, wg_ref[...], preferred_element_type=jnp.float32) + cg_ref[...]
    xr1_ref[...] = jnp.dot(xb, wr_ref[...], preferred_element_type=jnp.float32) + br_ref[...]


def _tc2_body(acc_ref, xr1_ref, wl_ref, wr_ref, b2_ref, g2_ref, hr2_ref, ic_ref):
    acc = acc_ref[...]
    s = acc[0] + acc[1]
    col = lax.broadcasted_iota(jnp.int32, s.shape, 1)
    cnt = jnp.sum(jnp.where(col == 40, s, 0.0), axis=1, keepdims=True)
    ic = 1.0 / jnp.maximum(cnt, 1.0)
    h = jnp.maximum(s * ic + xr1_ref[...], 0.0)
    g2_ref[...] = jnp.dot(h, wl_ref[...], preferred_element_type=jnp.float32)
    hr2_ref[...] = jnp.dot(h, wr_ref[...], preferred_element_type=jnp.float32) + b2_ref[...]
    ic_ref[...] = ic


def _tc3_body(acc_ref, hr2_ref, ic_ref, out_ref):
    acc = acc_ref[...]
    s = acc[0] + acc[1]
    z = s * ic_ref[...] + hr2_ref[...]
    col = lax.broadcasted_iota(jnp.int32, z.shape, 1)
    valid = col < 24
    m = jnp.max(jnp.where(valid, z, -jnp.inf), axis=1, keepdims=True)
    e = jnp.where(valid, jnp.exp(z - m), 0.0)
    lse = jnp.log(jnp.sum(e, axis=1, keepdims=True))
    out_ref[...] = z - m - lse


def _row_blocked(width):
    return pl.BlockSpec((_BLK, width), lambda i: (i, 0))


def _full(shape):
    return pl.BlockSpec(shape, lambda i: tuple(0 for _ in shape))


def kernel(x, edge_index, W1_l, b1, W1_r, W2_l, b2, W2_r):
    f32 = jnp.float32
    xp = jnp.pad(x.astype(f32), ((0, _NP - _N), (0, 0)))
    src = edge_index[0].astype(jnp.int32)
    dst = edge_index[1].astype(jnp.int32)
    npad = _EPAD - _E
    src_p = jnp.concatenate([src, jnp.zeros((npad,), jnp.int32)]).reshape(_NW, _NCHUNK, _CH)
    dst_p = jnp.concatenate([dst, jnp.full((npad,), _N, jnp.int32)]).reshape(_NW, _NCHUNK, _CH)

    w1l48 = jnp.pad(W1_l.astype(f32), ((0, 0), (0, 8)))
    c1 = jnp.zeros((1, 48), f32).at[0, 40].set(1.0)
    w1r48 = jnp.pad(W1_r.astype(f32), ((0, 0), (0, 8)))
    b1_48 = jnp.pad(b1.astype(f32), (0, 8)).reshape(1, 48)
    w2l = jnp.pad(W2_l.astype(f32), ((0, 8), (0, 8)))
    w2r = jnp.pad(W2_r.astype(f32), ((0, 8), (0, 8)))
    b2_32 = jnp.pad(b2.astype(f32), (0, 8)).reshape(1, 32)
    z48 = jnp.zeros((_ROWS_PER_TILE, 48), f32)
    z32 = jnp.zeros((_ROWS_PER_TILE, 32), f32)

    g1, xr1 = pl.pallas_call(
        _tc1_body,
        grid=(_GRID,),
        in_specs=[_row_blocked(_D_IN), _full((_D_IN, 48)), _full((1, 48)),
                  _full((_D_IN, 48)), _full((1, 48))],
        out_specs=[_row_blocked(48), _row_blocked(48)],
        out_shape=[jax.ShapeDtypeStruct((_NP, 48), f32),
                   jax.ShapeDtypeStruct((_NP, 48), f32)],
    )(xp, w1l48, c1, w1r48, b1_48)

    acc1 = _sc_scatter_fn(48, _NCHUNK)(g1, src_p, dst_p, z48)

    g2, hr2, ic = pl.pallas_call(
        _tc2_body,
        grid=(_GRID,),
        in_specs=[pl.BlockSpec((2, _BLK, 48), lambda i: (0, i, 0)),
                  _row_blocked(48), _full((48, 32)), _full((48, 32)), _full((1, 32))],
        out_specs=[_row_blocked(32), _row_blocked(32), _row_blocked(1)],
        out_shape=[jax.ShapeDtypeStruct((_NP, 32), f32),
                   jax.ShapeDtypeStruct((_NP, 32), f32),
                   jax.ShapeDtypeStruct((_NP, 1), f32)],
    )(acc1, xr1, w2l, w2r, b2_32)

    acc2 = _sc_scatter_fn(32, _NCHUNK)(g2, src_p, dst_p, z32)

    out = pl.pallas_call(
        _tc3_body,
        grid=(_GRID,),
        in_specs=[pl.BlockSpec((2, _BLK, 32), lambda i: (0, i, 0)),
                  _row_blocked(32), _row_blocked(1)],
        out_specs=_row_blocked(32),
        out_shape=jax.ShapeDtypeStruct((_NP, 32), f32),
    )(acc2, hr2, ic)

    return out[:_N, :24]


# speedup vs baseline: 1.7578x; 1.7578x over previous
import functools

import jax
import jax.numpy as jnp
from jax import lax
from jax.experimental import pallas as pl
from jax.experimental.pallas import tpu as pltpu
from jax.experimental.pallas import tpu_sc as plsc

_N = 10000
_E = 320000
_D_IN = 128

_NP = 10240
_ROWS_PER_TILE = _NP // 16
_NW = 32
_CH = 128
_NBUF = 2
_NCHUNK = _NBUF * (-(-_E // (_NW * _CH * _NBUF)))
_EPAD = _NW * _NCHUNK * _CH
_BLK = 1024
_GRID = _NP // _BLK


def _sc_scatter_fn(width, nchunk):
    mesh = plsc.VectorSubcoreMesh(
        core_axis_name="c", subcore_axis_name="s", num_cores=2, num_subcores=16
    )

    @functools.partial(
        pl.kernel,
        out_type=jax.ShapeDtypeStruct((2, _NP, width), jnp.float32),
        mesh=mesh,
        scratch_types=[
            pltpu.VMEM((nchunk, _CH), jnp.int32),
            pltpu.VMEM((nchunk, _CH), jnp.int32),
            pltpu.VMEM((_NBUF, _CH, width), jnp.float32),
            pltpu.VMEM_SHARED((_NP, width), jnp.float32),
            pltpu.VMEM_SHARED((_NP, width), jnp.float32),
        ] + [pltpu.SemaphoreType.DMA] * _NBUF,
        compiler_params=pltpu.CompilerParams(use_tc_tiling_on_sc=False),
    )
    def sc_kernel(g_hbm, src_hbm, dst_hbm, zeros_hbm, out_hbm,
                  src_v, dst_v, rows_v, acc_sh, tbl_sh, *sems):
        cid = lax.axis_index("c")
        sid = lax.axis_index("s")
        wid = sid * 2 + cid
        row0 = sid * _ROWS_PER_TILE
        pltpu.sync_copy(zeros_hbm, acc_sh.at[pl.ds(row0, _ROWS_PER_TILE)])
        pltpu.sync_copy(g_hbm.at[pl.ds(row0, _ROWS_PER_TILE)],
                        tbl_sh.at[pl.ds(row0, _ROWS_PER_TILE)])
        pltpu.sync_copy(src_hbm.at[wid], src_v)
        pltpu.sync_copy(dst_hbm.at[wid], dst_v)
        plsc.subcore_barrier()

        def body(j, carry):
            pltpu.async_copy(tbl_sh.at[src_v.at[j]], rows_v.at[0], sems[0]).wait()
            pltpu.sync_copy(rows_v.at[0], acc_sh.at[dst_v.at[j]], add=True)
            return carry

        lax.fori_loop(0, nchunk, body, 0)
        plsc.subcore_barrier()
        pltpu.sync_copy(acc_sh.at[pl.ds(row0, _ROWS_PER_TILE)],
                        out_hbm.at[cid, pl.ds(row0, _ROWS_PER_TILE)])

    return sc_kernel


def _tc1_body(x_ref, wg_ref, cg_ref, wr_ref, br_ref, g1_ref, xr1_ref):
    xb = x_ref[...]
    g1_ref[...] = jnp.dot(xb, wg_ref[...], preferred_element_type=jnp.float32) + cg_ref[...]
    xr1_ref[...] = jnp.dot(xb, wr_ref[...], preferred_element_type=jnp.float32) + br_ref[...]


def _tc2_body(acc_ref, xr1_ref, wl_ref, wr_ref, b2_ref, g2_ref, hr2_ref, ic_ref):
    acc = acc_ref[...]
    s = acc[0] + acc[1]
    col = lax.broadcasted_iota(jnp.int32, s.shape, 1)
    cnt = jnp.sum(jnp.where(col == 40, s, 0.0), axis=1, keepdims=True)
    ic = 1.0 / jnp.maximum(cnt, 1.0)
    h = jnp.maximum(s * ic + xr1_ref[...], 0.0)
    g2_ref[...] = jnp.dot(h, wl_ref[...], preferred_element_type=jnp.float32)
    hr2_ref[...] = jnp.dot(h, wr_ref[...], preferred_element_type=jnp.float32) + b2_ref[...]
    ic_ref[...] = ic


def _tc3_body(acc_ref, hr2_ref, ic_ref, out_ref):
    acc = acc_ref[...]
    s = acc[0] + acc[1]
    z = s * ic_ref[...] + hr2_ref[...]
    col = lax.broadcasted_iota(jnp.int32, z.shape, 1)
    valid = col < 24
    m = jnp.max(jnp.where(valid, z, -jnp.inf), axis=1, keepdims=True)
    e = jnp.where(valid, jnp.exp(z - m), 0.0)
    lse = jnp.log(jnp.sum(e, axis=1, keepdims=True))
    out_ref[...] = z - m - lse


def _row_blocked(width):
    return pl.BlockSpec((_BLK, width), lambda i: (i, 0))


def _full(shape):
    return pl.BlockSpec(shape, lambda i: tuple(0 for _ in shape))


def kernel(x, edge_index, W1_l, b1, W1_r, W2_l, b2, W2_r):
    f32 = jnp.float32
    xp = jnp.pad(x.astype(f32), ((0, _NP - _N), (0, 0)))
    src = edge_index[0].astype(jnp.int32)
    dst = edge_index[1].astype(jnp.int32)
    npad = _EPAD - _E
    src_p = jnp.concatenate([src, jnp.zeros((npad,), jnp.int32)]).reshape(_NW, _NCHUNK, _CH)
    dst_p = jnp.concatenate([dst, jnp.full((npad,), _N, jnp.int32)]).reshape(_NW, _NCHUNK, _CH)

    w1l48 = jnp.pad(W1_l.astype(f32), ((0, 0), (0, 8)))
    c1 = jnp.zeros((1, 48), f32).at[0, 40].set(1.0)
    w1r48 = jnp.pad(W1_r.astype(f32), ((0, 0), (0, 8)))
    b1_48 = jnp.pad(b1.astype(f32), (0, 8)).reshape(1, 48)
    w2l = jnp.pad(W2_l.astype(f32), ((0, 8), (0, 8)))
    w2r = jnp.pad(W2_r.astype(f32), ((0, 8), (0, 8)))
    b2_32 = jnp.pad(b2.astype(f32), (0, 8)).reshape(1, 32)
    z48 = jnp.zeros((_ROWS_PER_TILE, 48), f32)
    z32 = jnp.zeros((_ROWS_PER_TILE, 32), f32)

    g1, xr1 = pl.pallas_call(
        _tc1_body,
        grid=(_GRID,),
        in_specs=[_row_blocked(_D_IN), _full((_D_IN, 48)), _full((1, 48)),
                  _full((_D_IN, 48)), _full((1, 48))],
        out_specs=[_row_blocked(48), _row_blocked(48)],
        out_shape=[jax.ShapeDtypeStruct((_NP, 48), f32),
                   jax.ShapeDtypeStruct((_NP, 48), f32)],
    )(xp, w1l48, c1, w1r48, b1_48)

    acc1 = _sc_scatter_fn(48, _NCHUNK)(g1, src_p, dst_p, z48)

    g2, hr2, ic = pl.pallas_call(
        _tc2_body,
        grid=(_GRID,),
        in_specs=[pl.BlockSpec((2, _BLK, 48), lambda i: (0, i, 0)),
                  _row_blocked(48), _full((48, 32)), _full((48, 32)), _full((1, 32))],
        out_specs=[_row_blocked(32), _row_blocked(32), _row_blocked(1)],
        out_shape=[jax.ShapeDtypeStruct((_NP, 32), f32),
                   jax.ShapeDtypeStruct((_NP, 32), f32),
                   jax.ShapeDtypeStruct((_NP, 1), f32)],
    )(acc1, xr1, w2l, w2r, b2_32)

    acc2 = _sc_scatter_fn(32, _NCHUNK)(g2, src_p, dst_p, z32)

    out = pl.pallas_call(
        _tc3_body,
        grid=(_GRID,),
        in_specs=[pl.BlockSpec((2, _BLK, 32), lambda i: (0, i, 0)),
                  _row_blocked(32), _row_blocked(1)],
        out_specs=_row_blocked(32),
        out_shape=jax.ShapeDtypeStruct((_NP, 32), f32),
    )(acc2, hr2, ic)

    return out[:_N, :24]


# speedup vs baseline: 1.9944x; 1.1346x over previous
import functools

import jax
import jax.numpy as jnp
from jax import lax
from jax.experimental import pallas as pl
from jax.experimental.pallas import tpu as pltpu
from jax.experimental.pallas import tpu_sc as plsc

_N = 10000
_E = 320000
_D_IN = 128

_NP = 10240
_ROWS_PER_TILE = _NP // 16
_NW = 32
_CH = 128
_NBUF = 2
_NCHUNK = _NBUF * (-(-_E // (_NW * _CH * _NBUF)))
_EPAD = _NW * _NCHUNK * _CH
_BLK = 1024
_GRID = _NP // _BLK


def _sc_scatter_fn(width, nchunk):
    mesh = plsc.VectorSubcoreMesh(
        core_axis_name="c", subcore_axis_name="s", num_cores=2, num_subcores=16
    )

    @functools.partial(
        pl.kernel,
        out_type=jax.ShapeDtypeStruct((2, _NP, width), jnp.float32),
        mesh=mesh,
        scratch_types=[
            pltpu.VMEM((nchunk, _CH), jnp.int32),
            pltpu.VMEM((nchunk, _CH), jnp.int32),
            pltpu.VMEM((_NBUF, _CH, width), jnp.float32),
            pltpu.VMEM_SHARED((_NP, width), jnp.float32),
            pltpu.VMEM_SHARED((_NP, width), jnp.float32),
        ] + [pltpu.SemaphoreType.DMA] * _NBUF,
        compiler_params=pltpu.CompilerParams(use_tc_tiling_on_sc=False),
    )
    def sc_kernel(g_hbm, src_hbm, dst_hbm, zeros_hbm, out_hbm,
                  src_v, dst_v, rows_v, acc_sh, tbl_sh, *sems):
        cid = lax.axis_index("c")
        sid = lax.axis_index("s")
        wid = sid * 2 + cid
        row0 = sid * _ROWS_PER_TILE
        pltpu.sync_copy(zeros_hbm, acc_sh.at[pl.ds(row0, _ROWS_PER_TILE)])
        pltpu.sync_copy(g_hbm.at[pl.ds(row0, _ROWS_PER_TILE)],
                        tbl_sh.at[pl.ds(row0, _ROWS_PER_TILE)])
        pltpu.sync_copy(src_hbm.at[wid], src_v)
        pltpu.sync_copy(dst_hbm.at[wid], dst_v)
        plsc.subcore_barrier()

        def step(j, p, fire_next):
            pltpu.make_async_copy(tbl_sh.at[src_v.at[j]], rows_v.at[p],
                                  sems[p]).wait()
            if fire_next:
                q = 1 - p
                pltpu.async_copy(tbl_sh.at[src_v.at[j + 1]], rows_v.at[q],
                                 sems[q])
            pltpu.sync_copy(rows_v.at[p], acc_sh.at[dst_v.at[j]], add=True)

        pltpu.async_copy(tbl_sh.at[src_v.at[0]], rows_v.at[0], sems[0])

        def body(i, carry):
            step(i * 2, 0, True)
            step(i * 2 + 1, 1, True)
            return carry

        lax.fori_loop(0, nchunk // 2 - 1, body, 0)
        step(nchunk - 2, 0, True)
        step(nchunk - 1, 1, False)
        plsc.subcore_barrier()
        pltpu.sync_copy(acc_sh.at[pl.ds(row0, _ROWS_PER_TILE)],
                        out_hbm.at[cid, pl.ds(row0, _ROWS_PER_TILE)])

    return sc_kernel


def _tc1_body(x_ref, wg_ref, cg_ref, wr_ref, br_ref, g1_ref, xr1_ref):
    xb = x_ref[...]
    g1_ref[...] = jnp.dot(xb, wg_ref[...], preferred_element_type=jnp.float32) + cg_ref[...]
    xr1_ref[...] = jnp.dot(xb, wr_ref[...], preferred_element_type=jnp.float32) + br_ref[...]


def _tc2_body(acc_ref, xr1_ref, wl_ref, wr_ref, b2_ref, g2_ref, hr2_ref, ic_ref):
    acc = acc_ref[...]
    s = acc[0] + acc[1]
    col = lax.broadcasted_iota(jnp.int32, s.shape, 1)
    cnt = jnp.sum(jnp.where(col == 40, s, 0.0), axis=1, keepdims=True)
    ic = 1.0 / jnp.maximum(cnt, 1.0)
    h = jnp.maximum(s * ic + xr1_ref[...], 0.0)
    g2_ref[...] = jnp.dot(h, wl_ref[...], preferred_element_type=jnp.float32)
    hr2_ref[...] = jnp.dot(h, wr_ref[...], preferred_element_type=jnp.float32) + b2_ref[...]
    ic_ref[...] = ic


def _tc3_body(acc_ref, hr2_ref, ic_ref, out_ref):
    acc = acc_ref[...]
    s = acc[0] + acc[1]
    z = s * ic_ref[...] + hr2_ref[...]
    col = lax.broadcasted_iota(jnp.int32, z.shape, 1)
    valid = col < 24
    m = jnp.max(jnp.where(valid, z, -jnp.inf), axis=1, keepdims=True)
    e = jnp.where(valid, jnp.exp(z - m), 0.0)
    lse = jnp.log(jnp.sum(e, axis=1, keepdims=True))
    out_ref[...] = z - m - lse


def _row_blocked(width):
    return pl.BlockSpec((_BLK, width), lambda i: (i, 0))


def _full(shape):
    return pl.BlockSpec(shape, lambda i: tuple(0 for _ in shape))


def kernel(x, edge_index, W1_l, b1, W1_r, W2_l, b2, W2_r):
    f32 = jnp.float32
    xp = jnp.pad(x.astype(f32), ((0, _NP - _N), (0, 0)))
    src = edge_index[0].astype(jnp.int32)
    dst = edge_index[1].astype(jnp.int32)
    npad = _EPAD - _E
    src_p = jnp.concatenate([src, jnp.zeros((npad,), jnp.int32)]).reshape(_NW, _NCHUNK, _CH)
    dst_p = jnp.concatenate([dst, jnp.full((npad,), _N, jnp.int32)]).reshape(_NW, _NCHUNK, _CH)

    w1l48 = jnp.pad(W1_l.astype(f32), ((0, 0), (0, 8)))
    c1 = jnp.zeros((1, 48), f32).at[0, 40].set(1.0)
    w1r48 = jnp.pad(W1_r.astype(f32), ((0, 0), (0, 8)))
    b1_48 = jnp.pad(b1.astype(f32), (0, 8)).reshape(1, 48)
    w2l = jnp.pad(W2_l.astype(f32), ((0, 8), (0, 8)))
    w2r = jnp.pad(W2_r.astype(f32), ((0, 8), (0, 8)))
    b2_32 = jnp.pad(b2.astype(f32), (0, 8)).reshape(1, 32)
    z48 = jnp.zeros((_ROWS_PER_TILE, 48), f32)
    z32 = jnp.zeros((_ROWS_PER_TILE, 32), f32)

    g1, xr1 = pl.pallas_call(
        _tc1_body,
        grid=(_GRID,),
        in_specs=[_row_blocked(_D_IN), _full((_D_IN, 48)), _full((1, 48)),
                  _full((_D_IN, 48)), _full((1, 48))],
        out_specs=[_row_blocked(48), _row_blocked(48)],
        out_shape=[jax.ShapeDtypeStruct((_NP, 48), f32),
                   jax.ShapeDtypeStruct((_NP, 48), f32)],
    )(xp, w1l48, c1, w1r48, b1_48)

    acc1 = _sc_scatter_fn(48, _NCHUNK)(g1, src_p, dst_p, z48)

    g2, hr2, ic = pl.pallas_call(
        _tc2_body,
        grid=(_GRID,),
        in_specs=[pl.BlockSpec((2, _BLK, 48), lambda i: (0, i, 0)),
                  _row_blocked(48), _full((48, 32)), _full((48, 32)), _full((1, 32))],
        out_specs=[_row_blocked(32), _row_blocked(32), _row_blocked(1)],
        out_shape=[jax.ShapeDtypeStruct((_NP, 32), f32),
                   jax.ShapeDtypeStruct((_NP, 32), f32),
                   jax.ShapeDtypeStruct((_NP, 1), f32)],
    )(acc1, xr1, w2l, w2r, b2_32)

    acc2 = _sc_scatter_fn(32, _NCHUNK)(g2, src_p, dst_p, z32)

    out = pl.pallas_call(
        _tc3_body,
        grid=(_GRID,),
        in_specs=[pl.BlockSpec((2, _BLK, 32), lambda i: (0, i, 0)),
                  _row_blocked(32), _row_blocked(1)],
        out_specs=_row_blocked(32),
        out_shape=jax.ShapeDtypeStruct((_NP, 32), f32),
    )(acc2, hr2, ic)

    return out[:_N, :24]
